# baseline (device time: 14697 ns/iter reference)
import jax
import jax.numpy as jnp
from jax import lax
from jax.experimental import pallas as pl
from jax.experimental.pallas import tpu as pltpu

N_DEV = 16


def kernel(x, w_mat):
    m_per, k = x.shape
    _, n = w_mat.shape
    n_per = n // N_DEV
    n_chunks = N_DEV // 2
    assert n % N_DEV == 0

    def body(x_hbm, w_hbm, out_ref, xv, wbuf, y_ref,
             x_sem, w_sems, send_sems, recv_sems):
        my = lax.axis_index("i")

        barrier = pltpu.get_barrier_semaphore()
        for d in (1, N_DEV - 1):
            peer = lax.rem(my + d, N_DEV)
            pl.semaphore_signal(
                barrier, inc=1,
                device_id=(peer,), device_id_type=pl.DeviceIdType.MESH,
            )

        cp_x = pltpu.make_async_copy(x_hbm, xv, x_sem.at[0])
        cp_x.start()
        cp_w = []
        for t in range(n_chunks):
            cp = pltpu.make_async_copy(
                w_hbm.at[:, 2 * t * n_per:(2 * t + 2) * n_per],
                wbuf.at[t],
                w_sems.at[t],
            )
            cp.start()
            cp_w.append(cp)

        cp_x.wait()
        xb = xv[:, :].astype(jnp.bfloat16)
        for t in range(n_chunks):
            cp_w[t].wait()
            wb = wbuf[t, :, :].astype(jnp.bfloat16)
            pair = jnp.dot(xb, wb, preferred_element_type=jnp.float32)
            pair = jnp.maximum(pair, 0.0).astype(jnp.bfloat16)
            y_ref[2 * t, :, :] = pair[:, :n_per]
            y_ref[2 * t + 1, :, :] = pair[:, n_per:]

        out_ref[pl.ds(my * m_per, m_per), :] = y_ref[my, :, :]

        pl.semaphore_wait(barrier, 2)

        send_rdmas = []
        for d in range(1, N_DEV):
            peer = lax.rem(my + d, N_DEV)
            rdma = pltpu.make_async_remote_copy(
                src_ref=y_ref.at[peer],
                dst_ref=out_ref.at[pl.ds(my * m_per, m_per), :],
                send_sem=send_sems.at[d],
                recv_sem=recv_sems.at[d],
                device_id=(peer,),
                device_id_type=pl.DeviceIdType.MESH,
            )
            rdma.start()
            send_rdmas.append(rdma)

        for rdma in send_rdmas:
            rdma.wait_send()

        for d in range(1, N_DEV):
            src = lax.rem(my + N_DEV - d, N_DEV)
            recv = pltpu.make_async_remote_copy(
                src_ref=y_ref.at[src],
                dst_ref=out_ref.at[pl.ds(src * m_per, m_per), :],
                send_sem=send_sems.at[d],
                recv_sem=recv_sems.at[d],
                device_id=(src,),
                device_id_type=pl.DeviceIdType.MESH,
            )
            recv.wait_recv()

    return pl.pallas_call(
        body,
        out_shape=jax.ShapeDtypeStruct((N_DEV * m_per, n_per), jnp.bfloat16),
        in_specs=[
            pl.BlockSpec(memory_space=pl.ANY),
            pl.BlockSpec(memory_space=pl.ANY),
        ],
        out_specs=pl.BlockSpec(memory_space=pltpu.VMEM),
        scratch_shapes=[
            pltpu.VMEM((m_per, k), jnp.float32),
            pltpu.VMEM((n_chunks, k, 2 * n_per), jnp.float32),
            pltpu.VMEM((N_DEV, m_per, n_per), jnp.bfloat16),
            pltpu.SemaphoreType.DMA((1,)),
            pltpu.SemaphoreType.DMA((n_chunks,)),
            pltpu.SemaphoreType.DMA((N_DEV,)),
            pltpu.SemaphoreType.DMA((N_DEV,)),
        ],
        compiler_params=pltpu.CompilerParams(collective_id=0),
    )(x, w_mat)


# device time: 12968 ns/iter; 1.1333x vs baseline; 1.1333x over previous
import jax
import jax.numpy as jnp
from jax import lax
from jax.experimental import pallas as pl
from jax.experimental.pallas import tpu as pltpu

N_DEV = 16


def kernel(x, w_mat):
    m_per, k = x.shape
    _, n = w_mat.shape
    n_per = n // N_DEV
    assert n % N_DEV == 0

    def body(x_ref, w_ref, out_ref, y_ref, send_sems, recv_sems):
        my = lax.axis_index("i")

        barrier = pltpu.get_barrier_semaphore()
        for d in (1, N_DEV - 1):
            peer = lax.rem(my + d, N_DEV)
            pl.semaphore_signal(
                barrier, inc=1,
                device_id=(peer,), device_id_type=pl.DeviceIdType.MESH,
            )

        xb = x_ref[:, :].astype(jnp.bfloat16)

        send_rdmas = []
        for t in range(N_DEV // 2):
            wb = w_ref[:, 2 * t * n_per:(2 * t + 2) * n_per].astype(
                jnp.bfloat16
            )
            pair = jnp.dot(xb, wb, preferred_element_type=jnp.float32)
            pair = jnp.maximum(pair, 0.0).astype(jnp.bfloat16)
            y_ref[2 * t, :, :] = pair[:, :n_per]
            y_ref[2 * t + 1, :, :] = pair[:, n_per:]
            if t == 0:
                pl.semaphore_wait(barrier, 2)
            for j in (2 * t, 2 * t + 1):
                rdma = pltpu.make_async_remote_copy(
                    src_ref=y_ref.at[j],
                    dst_ref=out_ref.at[pl.ds(my * m_per, m_per), :],
                    send_sem=send_sems.at[j],
                    recv_sem=recv_sems.at[my],
                    device_id=(j,),
                    device_id_type=pl.DeviceIdType.MESH,
                )

                @pl.when(j != my)
                def _(rdma=rdma):
                    rdma.start()

                send_rdmas.append((j, rdma))

        out_ref[pl.ds(my * m_per, m_per), :] = y_ref[my, :, :]

        for j, rdma in send_rdmas:
            @pl.when(j != my)
            def _(rdma=rdma):
                rdma.wait_send()

        for s in range(N_DEV):
            recv = pltpu.make_async_remote_copy(
                src_ref=y_ref.at[s],
                dst_ref=out_ref.at[pl.ds(s * m_per, m_per), :],
                send_sem=send_sems.at[s],
                recv_sem=recv_sems.at[s],
                device_id=(s,),
                device_id_type=pl.DeviceIdType.MESH,
            )

            @pl.when(s != my)
            def _(recv=recv):
                recv.wait_recv()

    return pl.pallas_call(
        body,
        out_shape=jax.ShapeDtypeStruct((N_DEV * m_per, n_per), jnp.bfloat16),
        in_specs=[
            pl.BlockSpec(memory_space=pltpu.VMEM),
            pl.BlockSpec(memory_space=pltpu.VMEM),
        ],
        out_specs=pl.BlockSpec(memory_space=pltpu.VMEM),
        scratch_shapes=[
            pltpu.VMEM((N_DEV, m_per, n_per), jnp.bfloat16),
            pltpu.SemaphoreType.DMA((N_DEV,)),
            pltpu.SemaphoreType.DMA((N_DEV,)),
        ],
        compiler_params=pltpu.CompilerParams(collective_id=0),
    )(x, w_mat)


# device time: 12493 ns/iter; 1.1764x vs baseline; 1.0380x over previous
import jax
import jax.numpy as jnp
from jax import lax
from jax.experimental import pallas as pl
from jax.experimental.pallas import tpu as pltpu

N_DEV = 16


def kernel(x, w_mat):
    m_per, k = x.shape
    _, n = w_mat.shape
    n_per = n // N_DEV
    assert n % N_DEV == 0

    def body(x_ref, w_ref, out_ref, y_ref, send_sems, recv_sems):
        my = lax.axis_index("i")

        barrier = pltpu.get_barrier_semaphore()
        for d in (1, N_DEV - 1):
            peer = lax.rem(my + d, N_DEV)
            pl.semaphore_signal(
                barrier, inc=1,
                device_id=(peer,), device_id_type=pl.DeviceIdType.MESH,
            )

        xb = x_ref[:, :].astype(jnp.bfloat16)
        for t in range(N_DEV // 4):
            wb = w_ref[:, 4 * t * n_per:(4 * t + 4) * n_per].astype(
                jnp.bfloat16
            )
            quad = jnp.dot(xb, wb, preferred_element_type=jnp.float32)
            quad = jnp.maximum(quad, 0.0).astype(jnp.bfloat16)
            for q in range(4):
                y_ref[4 * t + q, :, :] = quad[:, q * n_per:(q + 1) * n_per]

        out_ref[pl.ds(my * m_per, m_per), :] = y_ref[my, :, :]

        pl.semaphore_wait(barrier, 2)

        send_rdmas = []
        for d in range(1, N_DEV):
            peer = lax.rem(my + d, N_DEV)
            rdma = pltpu.make_async_remote_copy(
                src_ref=y_ref.at[peer],
                dst_ref=out_ref.at[pl.ds(my * m_per, m_per), :],
                send_sem=send_sems.at[d],
                recv_sem=recv_sems.at[d],
                device_id=(peer,),
                device_id_type=pl.DeviceIdType.MESH,
            )
            rdma.start()
            send_rdmas.append(rdma)

        for d in range(1, N_DEV):
            src = lax.rem(my + N_DEV - d, N_DEV)
            recv = pltpu.make_async_remote_copy(
                src_ref=y_ref.at[src],
                dst_ref=out_ref.at[pl.ds(src * m_per, m_per), :],
                send_sem=send_sems.at[d],
                recv_sem=recv_sems.at[d],
                device_id=(src,),
                device_id_type=pl.DeviceIdType.MESH,
            )
            recv.wait_recv()

        for rdma in send_rdmas:
            rdma.wait_send()

    return pl.pallas_call(
        body,
        out_shape=jax.ShapeDtypeStruct((N_DEV * m_per, n_per), jnp.bfloat16),
        in_specs=[
            pl.BlockSpec(memory_space=pltpu.VMEM),
            pl.BlockSpec(memory_space=pltpu.VMEM),
        ],
        out_specs=pl.BlockSpec(memory_space=pltpu.VMEM),
        scratch_shapes=[
            pltpu.VMEM((N_DEV, m_per, n_per), jnp.bfloat16),
            pltpu.SemaphoreType.DMA((N_DEV,)),
            pltpu.SemaphoreType.DMA((N_DEV,)),
        ],
        compiler_params=pltpu.CompilerParams(collective_id=0),
    )(x, w_mat)
